# Initial kernel scaffold; baseline (speedup 1.0000x reference)
import jax
import jax.numpy as jnp
from jax.experimental import pallas as pl


def kernel(x, conv1_w, conv1_b, conv2_w, conv2_b, conv3_w, conv3_b, fc1_w, fc1_b, fc2_w, fc2_b):
    raise NotImplementedError("write your pallas kernel here")


# speedup vs baseline: 1.3922x; 1.3922x over previous
import numpy as np
import jax
import jax.numpy as jnp
from jax.experimental import pallas as pl
from jax.experimental.pallas import tpu as pltpu

BOARD = 19
HW = BOARD * BOARD
CH = 32

K1, P1 = 9, 4
K2, P2 = 7, 3
K3, P3 = 5, 2
S27 = BOARD + 2 * P1
PB = 768
LEAD = 16
VB = LEAD + P1 * S27 + P1
LV = (BOARD - 1) * S27 + BOARD
FW = 512
BT = 8
XW = BT * PB
ACCW = (BT - 1) * PB + FW
SW = ACCW + 8

KFEAT = CH * FW
NOUT = 384
BH = 256

_CONV_VMEM = 32 * 1024 * 1024
_HEAD_VMEM = 50 * 1024 * 1024


def _col_mask():
    p = np.arange(ACCW)
    r = p % PB
    valid = (r < LV) & ((r % S27) < BOARD)
    return jnp.asarray(valid.astype(np.float32)[None, :])


def _conv_kernel(x_ref, w1_ref, b1_ref, w2_ref, b2_ref, w3_ref, b3_ref,
                 mask_ref, o_ref, buf_ref, stg_ref):
    mask = mask_ref[...]

    buf_ref[...] = jnp.zeros_like(buf_ref)

    for dy in range(K1):
        s = LEAD + dy * S27
        stg_ref[208 + dy, :] = x_ref[0, 0, s:s + SW].astype(jnp.bfloat16)
    for dx in range(K1):
        stg_ref[dx * K1:(dx + 1) * K1, :ACCW] = stg_ref[208:208 + K1,
                                                        dx:dx + ACCW]
    acc = jnp.dot(w1_ref[...], stg_ref[0:K1 * K1, 0:ACCW],
                  preferred_element_type=jnp.float32)
    a = jnp.maximum(acc + b1_ref[...], 0.0) * mask
    buf_ref[:, VB:VB + ACCW] = a.astype(jnp.bfloat16)

    for dy in range(K2):
        s = VB - P2 + (dy - P2) * S27
        stg_ref[dy * CH:(dy + 1) * CH, 0:ACCW + 2 * P2] = \
            buf_ref[:, s:s + ACCW + 2 * P2]
    acc = jnp.dot(w2_ref[0], stg_ref[0:K2 * CH, 0:ACCW],
                  preferred_element_type=jnp.float32)
    for dx in range(1, K2):
        acc = acc + jnp.dot(w2_ref[dx], stg_ref[0:K2 * CH, dx:dx + ACCW],
                            preferred_element_type=jnp.float32)
    a = jnp.maximum(acc + b2_ref[...], 0.0) * mask
    buf_ref[:, VB:VB + ACCW] = a.astype(jnp.bfloat16)

    for dy in range(K3):
        s = VB - P3 + (dy - P3) * S27
        stg_ref[dy * CH:(dy + 1) * CH, 0:ACCW + 2 * P3] = \
            buf_ref[:, s:s + ACCW + 2 * P3]
    acc = jnp.dot(w3_ref[0], stg_ref[0:K3 * CH, 0:ACCW],
                  preferred_element_type=jnp.float32)
    for dx in range(1, K3):
        acc = acc + jnp.dot(w3_ref[dx], stg_ref[0:K3 * CH, dx:dx + ACCW],
                            preferred_element_type=jnp.float32)
    a = jnp.maximum(acc + b3_ref[...], 0.0).astype(jnp.bfloat16)
    for b in range(BT):
        o_ref[b, :, :] = a[:, b * PB:b * PB + FW]


def _conv_stack(xg, w1f, b1, w2f, b2, w3f, b3):
    nb = xg.shape[0]
    return pl.pallas_call(
        _conv_kernel,
        out_shape=jax.ShapeDtypeStruct((nb * BT, CH, FW), jnp.bfloat16),
        grid=(nb,),
        in_specs=[
            pl.BlockSpec((1, 1, XW), lambda i: (i, 0, 0)),
            pl.BlockSpec((CH, K1 * K1), lambda i: (0, 0)),
            pl.BlockSpec((CH, 1), lambda i: (0, 0)),
            pl.BlockSpec((K2, CH, K2 * CH), lambda i: (0, 0, 0)),
            pl.BlockSpec((CH, 1), lambda i: (0, 0)),
            pl.BlockSpec((K3, CH, K3 * CH), lambda i: (0, 0, 0)),
            pl.BlockSpec((CH, 1), lambda i: (0, 0)),
            pl.BlockSpec((1, ACCW), lambda i: (0, 0)),
        ],
        out_specs=pl.BlockSpec((BT, CH, FW), lambda i: (i, 0, 0)),
        scratch_shapes=[
            pltpu.VMEM((CH, XW), jnp.bfloat16),
            pltpu.VMEM((224, SW), jnp.bfloat16),
        ],
        compiler_params=pltpu.CompilerParams(
            dimension_semantics=("parallel",),
            vmem_limit_bytes=_CONV_VMEM,
        ),
    )(xg, w1f, b1, w2f, b2, w3f, b3, _col_mask())


def _head_kernel(feat_ref, w1_ref, b1_ref, w2_ref, b2_ref, o_ref):
    h = jnp.dot(feat_ref[...], w1_ref[...],
                preferred_element_type=jnp.float32)
    h = jnp.maximum(h + b1_ref[...], 0.0).astype(jnp.bfloat16)
    y = jnp.dot(h, w2_ref[...], preferred_element_type=jnp.float32)
    y = y + b2_ref[...]
    m = jnp.max(y, axis=-1, keepdims=True)
    z = y - m
    lse = jnp.log(jnp.sum(jnp.exp(z), axis=-1, keepdims=True))
    o_ref[...] = z - lse


def _fc_head(feat, w1, b1, w2, b2):
    B = feat.shape[0]
    Bh = ((B + BH - 1) // BH) * BH
    if Bh != B:
        feat = jnp.pad(feat, ((0, Bh - B), (0, 0)))
    return pl.pallas_call(
        _head_kernel,
        out_shape=jax.ShapeDtypeStruct((Bh, NOUT), jnp.float32),
        grid=(Bh // BH,),
        in_specs=[
            pl.BlockSpec((BH, KFEAT), lambda i: (i, 0)),
            pl.BlockSpec((KFEAT, NOUT), lambda i: (0, 0)),
            pl.BlockSpec((1, NOUT), lambda i: (0, 0)),
            pl.BlockSpec((NOUT, NOUT), lambda i: (0, 0)),
            pl.BlockSpec((1, NOUT), lambda i: (0, 0)),
        ],
        out_specs=pl.BlockSpec((BH, NOUT), lambda i: (i, 0)),
        compiler_params=pltpu.CompilerParams(
            dimension_semantics=("parallel",),
            vmem_limit_bytes=_HEAD_VMEM,
        ),
    )(feat, w1, b1, w2, b2)


def _pack_input(x):
    B = x.shape[0]
    Bp = ((B + BT - 1) // BT) * BT
    xb = x.reshape(B, BOARD, BOARD)
    if Bp != B:
        xb = jnp.pad(xb, ((0, Bp - B), (0, 0), (0, 0)))
    xg = jnp.pad(xb, ((0, 0), (P1, P1), (P1, P1))).reshape(Bp, S27 * S27)
    xg = jnp.pad(xg, ((0, 0), (LEAD, PB - S27 * S27 - LEAD)))
    return xg.reshape(Bp // BT, 1, XW), Bp


def kernel(x, conv1_w, conv1_b, conv2_w, conv2_b, conv3_w, conv3_b,
           fc1_w, fc1_b, fc2_w, fc2_b):
    w1f = jnp.transpose(conv1_w, (1, 0, 2)).reshape(CH, K1 * K1)
    w1f = w1f.astype(jnp.bfloat16)
    w2f = jnp.transpose(conv2_w.reshape(K2, K2, CH, CH),
                        (1, 2, 0, 3)).reshape(K2, CH, K2 * CH)
    w3f = jnp.transpose(conv3_w.reshape(K3, K3, CH, CH),
                        (1, 2, 0, 3)).reshape(K3, CH, K3 * CH)

    B = x.shape[0]
    xg, Bp = _pack_input(x.astype(jnp.float32))
    feat = _conv_stack(xg, w1f, conv1_b, w2f, conv2_b, w3f, conv3_b)
    feat = feat.reshape(Bp, KFEAT)
    out = _fc_head(feat, fc1_w, fc1_b, fc2_w, fc2_b)
    return out[:B, :HW]


# speedup vs baseline: 2.9318x; 2.1059x over previous
import numpy as np
import jax
import jax.numpy as jnp
from jax.experimental import pallas as pl
from jax.experimental.pallas import tpu as pltpu

BOARD = 19
HW = BOARD * BOARD
CH = 32

K1, P1 = 9, 4
K2, P2 = 7, 3
K3, P3 = 5, 2
S27 = BOARD + 2 * P1
PB = 768
LEAD = 16
VB = LEAD + P1 * S27 + P1
LV = (BOARD - 1) * S27 + BOARD
FW = 512
BT = 32
XW = BT * PB
ACCW = (BT - 1) * PB + FW
SW = ACCW + 8
CW = 3072
CW2 = 1536

KFEAT = CH * FW
NOUT = 384
BH = 256

_CONV_VMEM = 48 * 1024 * 1024
_HEAD_VMEM = 50 * 1024 * 1024

def _chunks(cw):
    return [(s, min(cw, ACCW - s)) for s in range(0, ACCW, cw)]


def _col_mask():
    p = np.arange(ACCW)
    r = p % PB
    valid = (r < LV) & ((r % S27) < BOARD)
    return jnp.asarray(valid.astype(np.float32)[None, :])


def _conv_kernel(x_ref, w1_ref, b1_ref, w2_ref, b2_ref, w3_ref, b3_ref,
                 mask_ref, o_ref, buf_ref, stg_ref, pp_ref):
    mask = mask_ref[...]
    f32 = jnp.float32

    buf_ref[...] = jnp.zeros_like(buf_ref)

    for dy in range(K1):
        s = LEAD + dy * S27
        stg_ref[208 + dy, :] = x_ref[0, 0, s:s + SW]
    for dx in range(K1):
        stg_ref[dx * K1:(dx + 1) * K1, :ACCW] = stg_ref[208:208 + K1,
                                                        dx:dx + ACCW]
    pp_ref[0:CH, 0:ACCW] = jnp.dot(
        w1_ref[...], stg_ref[0:K1 * K1, 0:ACCW],
        preferred_element_type=f32).astype(jnp.bfloat16)
    for s, w in _chunks(CW):
        a = jnp.maximum(pp_ref[0:CH, s:s + w].astype(f32)
                        + b1_ref[...], 0.0) * mask[:, s:s + w]
        buf_ref[:, VB + s:VB + s + w] = a.astype(jnp.bfloat16)

    W2 = ACCW + 2 * P2
    for dy in range(K2):
        s = VB - P2 + (dy - P2) * S27
        stg_ref[dy * CH:(dy + 1) * CH, 0:W2] = buf_ref[:, s:s + W2]
    pp_ref[0:K2 * CH, 0:W2] = jnp.dot(
        w2_ref[...], stg_ref[0:K2 * CH, 0:W2],
        preferred_element_type=f32).astype(jnp.bfloat16)
    for s, w in _chunks(CW):
        acc = pp_ref[0:CH, s:s + w].astype(f32)
        for dx in range(1, K2):
            acc = acc + pp_ref[dx * CH:(dx + 1) * CH,
                               s + dx:s + dx + w].astype(f32)
        a = jnp.maximum(acc + b2_ref[...], 0.0) * mask[:, s:s + w]
        buf_ref[:, VB + s:VB + s + w] = a.astype(jnp.bfloat16)

    W3 = ACCW + 2 * P3
    for dy in range(K3):
        s = VB - P3 + (dy - P3) * S27
        stg_ref[dy * CH:(dy + 1) * CH, 0:W3] = buf_ref[:, s:s + W3]
    pp_ref[0:K3 * CH, 0:W3] = jnp.dot(
        w3_ref[...], stg_ref[0:K3 * CH, 0:W3],
        preferred_element_type=f32).astype(jnp.bfloat16)
    for b in range(BT):
        q = b * PB
        acc = pp_ref[0:CH, q:q + FW].astype(f32)
        for dx in range(1, K3):
            acc = acc + pp_ref[dx * CH:(dx + 1) * CH,
                               q + dx:q + dx + FW].astype(f32)
        a = jnp.maximum(acc + b3_ref[...], 0.0)
        o_ref[b, :, :] = a.astype(jnp.bfloat16)


def _conv_stack(xg, w1f, b1, w2f, b2, w3f, b3):
    nb = xg.shape[0]
    return pl.pallas_call(
        _conv_kernel,
        out_shape=jax.ShapeDtypeStruct((nb * BT, CH, FW), jnp.bfloat16),
        grid=(nb,),
        in_specs=[
            pl.BlockSpec((1, 1, XW), lambda i: (i, 0, 0)),
            pl.BlockSpec((CH, K1 * K1), lambda i: (0, 0)),
            pl.BlockSpec((CH, 1), lambda i: (0, 0)),
            pl.BlockSpec((K2 * CH, K2 * CH), lambda i: (0, 0)),
            pl.BlockSpec((CH, 1), lambda i: (0, 0)),
            pl.BlockSpec((K3 * CH, K3 * CH), lambda i: (0, 0)),
            pl.BlockSpec((CH, 1), lambda i: (0, 0)),
            pl.BlockSpec((1, ACCW), lambda i: (0, 0)),
        ],
        out_specs=pl.BlockSpec((BT, CH, FW), lambda i: (i, 0, 0)),
        scratch_shapes=[
            pltpu.VMEM((CH, XW), jnp.bfloat16),
            pltpu.VMEM((224, SW), jnp.bfloat16),
            pltpu.VMEM((224, SW), jnp.bfloat16),
        ],
        compiler_params=pltpu.CompilerParams(
            dimension_semantics=("parallel",),
            vmem_limit_bytes=_CONV_VMEM,
        ),
    )(xg, w1f, b1, w2f, b2, w3f, b3, _col_mask())


def _head_kernel(feat_ref, w1_ref, b1_ref, w2_ref, b2_ref, o_ref):
    h = jnp.dot(feat_ref[...], w1_ref[...],
                preferred_element_type=jnp.float32)
    h = jnp.maximum(h + b1_ref[...], 0.0).astype(jnp.bfloat16)
    y = jnp.dot(h, w2_ref[...], preferred_element_type=jnp.float32)
    y = y + b2_ref[...]
    m = jnp.max(y, axis=-1, keepdims=True)
    z = y - m
    lse = jnp.log(jnp.sum(jnp.exp(z), axis=-1, keepdims=True))
    o_ref[...] = z - lse


def _fc_head(feat, w1, b1, w2, b2):
    B = feat.shape[0]
    Bh = ((B + BH - 1) // BH) * BH
    if Bh != B:
        feat = jnp.pad(feat, ((0, Bh - B), (0, 0)))
    return pl.pallas_call(
        _head_kernel,
        out_shape=jax.ShapeDtypeStruct((Bh, NOUT), jnp.float32),
        grid=(Bh // BH,),
        in_specs=[
            pl.BlockSpec((BH, KFEAT), lambda i: (i, 0)),
            pl.BlockSpec((KFEAT, NOUT), lambda i: (0, 0)),
            pl.BlockSpec((1, NOUT), lambda i: (0, 0)),
            pl.BlockSpec((NOUT, NOUT), lambda i: (0, 0)),
            pl.BlockSpec((1, NOUT), lambda i: (0, 0)),
        ],
        out_specs=pl.BlockSpec((BH, NOUT), lambda i: (i, 0)),
        compiler_params=pltpu.CompilerParams(
            dimension_semantics=("parallel",),
            vmem_limit_bytes=_HEAD_VMEM,
        ),
    )(feat, w1, b1, w2, b2)


def _pack_input(x):
    B = x.shape[0]
    Bp = ((B + BT - 1) // BT) * BT
    xb = x.reshape(B, BOARD, BOARD).astype(jnp.bfloat16)
    if Bp != B:
        xb = jnp.pad(xb, ((0, Bp - B), (0, 0), (0, 0)))
    xg = jnp.pad(xb, ((0, 0), (P1, P1), (P1, P1))).reshape(Bp, S27 * S27)
    xg = jnp.pad(xg, ((0, 0), (LEAD, PB - S27 * S27 - LEAD)))
    return xg.reshape(Bp // BT, 1, XW), Bp


def kernel(x, conv1_w, conv1_b, conv2_w, conv2_b, conv3_w, conv3_b,
           fc1_w, fc1_b, fc2_w, fc2_b):
    w1f = jnp.transpose(conv1_w, (1, 0, 2)).reshape(CH, K1 * K1)
    w1f = w1f.astype(jnp.bfloat16)
    w2f = jnp.transpose(conv2_w.reshape(K2, K2, CH, CH),
                        (1, 2, 0, 3)).reshape(K2 * CH, K2 * CH)
    w3f = jnp.transpose(conv3_w.reshape(K3, K3, CH, CH),
                        (1, 2, 0, 3)).reshape(K3 * CH, K3 * CH)

    B = x.shape[0]
    xg, Bp = _pack_input(x.astype(jnp.float32))
    feat = _conv_stack(xg, w1f, conv1_b, w2f, conv2_b, w3f, conv3_b)
    feat = feat.reshape(Bp, KFEAT)
    out = _fc_head(feat, fc1_w, fc1_b, fc2_w, fc2_b)
    return out[:B, :HW]
